# baseline (speedup 1.0000x reference)
import functools
import jax
import jax.numpy as jnp
from jax import lax
from jax.experimental import pallas as pl
from jax.experimental.pallas import tpu as pltpu
from jax.experimental.pallas import tpu_sc as plsc

F = 768
H = 64
E = 64
N = 2048
BT = 64
MAX_TILES = 96
TE_LEN = 128
PADDED = MAX_TILES * BT


def _router_body(x_ref, rs_ref, rl_ref, dst_ref, te_ref, xb_ref):
    xv = x_ref[...]
    xb_ref[...] = xv.astype(jnp.bfloat16)
    var = jnp.mean(xv * xv, axis=1, keepdims=True)
    ri = xv * lax.rsqrt(var + 1e-6)
    ri = ri * lax.rsqrt(jnp.float32(F)) * rs_ref[...]
    logits = jnp.dot(ri, rl_ref[...], preferred_element_type=jnp.float32)
    lane = lax.broadcasted_iota(jnp.int32, logits.shape, 1)
    maxv = jnp.max(logits, axis=1, keepdims=True)
    eid = jnp.min(jnp.where(logits == maxv, lane, E), axis=1)
    onehot = (eid[:, None] == lane).astype(jnp.float32)
    CH = 128
    NC = N // CH
    row = lax.broadcasted_iota(jnp.int32, (CH, CH), 0)
    col = lax.broadcasted_iota(jnp.int32, (CH, CH), 1)
    ltri = (row >= col).astype(jnp.float32)
    chunk_cums = []
    chunk_tot = []
    for ci in range(NC):
        blk = onehot[ci * CH:(ci + 1) * CH, :]
        cc = lax.dot_general(ltri, blk, (((1,), (0,)), ((), ())),
                             preferred_element_type=jnp.float32)
        chunk_cums.append(cc)
        chunk_tot.append(cc[CH - 1:CH, :])
    tot = jnp.concatenate(chunk_tot, axis=0)
    inc = tot
    k = 1
    while k < NC:
        inc = inc + jnp.concatenate(
            [jnp.zeros((k, E), jnp.float32), inc[: NC - k]], axis=0)
        k *= 2
    excl = inc - tot
    c = jnp.concatenate(
        [chunk_cums[ci] + excl[ci:ci + 1, :] for ci in range(NC)], axis=0)
    c = c.astype(jnp.int32)
    onehot = onehot.astype(jnp.int32)
    counts = c[N - 1 : N, :]
    rank = jnp.sum(onehot * c, axis=1) - 1
    pc = ((counts + BT - 1) // BT) * BT
    pci = pc
    k = 1
    while k < E:
        pci = pci + jnp.concatenate(
            [jnp.zeros((1, k), jnp.int32), pci[:, : E - k]], axis=1)
        k *= 2
    po = pci - pc
    dst = jnp.sum(onehot * po, axis=1) + rank
    dst_ref[...] = dst.reshape(N // 128, 128)
    starts = lax.broadcasted_iota(jnp.int32, (TE_LEN, E), 0) * BT
    te = jnp.sum((pci <= starts).astype(jnp.int32), axis=1)
    te = jnp.minimum(te, E - 1).reshape(1, TE_LEN)
    total = jnp.sum(jnp.where(lane[:1, :] == E - 1, pci, 0))
    n_used = total // BT
    meta_idx = lax.broadcasted_iota(jnp.int32, (1, TE_LEN), 1)
    te_shift = jnp.concatenate([te[:, -1:], te[:, :-1]], axis=1)
    te_ref[...] = jnp.where(meta_idx == 0, n_used, te_shift)


def _ffn_tile(xt, gw, lin, e, sc_ref):
    xb = xt.astype(jnp.bfloat16)
    h = lax.dot_general(xb, gw.astype(jnp.bfloat16),
                        (((1,), (1,)), ((), ())),
                        preferred_element_type=jnp.float32)
    lane = lax.broadcasted_iota(jnp.int32, (1, E), 1)
    scale = jnp.sum(jnp.where(lane == e, sc_ref[...], 0.0))
    act = jax.nn.gelu(h[:, :H], approximate=True) * h[:, H:] * scale
    return lax.dot_general(act.astype(jnp.bfloat16), lin.astype(jnp.bfloat16),
                           (((1,), (0,)), ((), ())),
                           preferred_element_type=jnp.float32)


TPS = 8


def _ffn_body(meta_ref, xs_ref, *refs):
    gw_refs = refs[:TPS]
    lin_refs = refs[TPS:2 * TPS]
    sc_ref = refs[2 * TPS]
    ys_ref = refs[2 * TPS + 1]
    ja = TPS * pl.program_id(0)
    n_used = meta_ref[0]

    @pl.when(ja < n_used)
    def _():
        lane = lax.broadcasted_iota(jnp.int32, (1, E), 1)
        hs = []
        for k in range(TPS):
            xb = xs_ref[k * BT:(k + 1) * BT, :]
            hs.append(lax.dot_general(
                xb, gw_refs[k][0].astype(jnp.bfloat16),
                (((1,), (1,)), ((), ())),
                preferred_element_type=jnp.float32))
        acts = []
        for k in range(TPS):
            e = meta_ref[jnp.minimum(ja + k, n_used - 1) + 1]
            scale = jnp.sum(jnp.where(lane == e, sc_ref[...], 0.0))
            h = hs[k]
            acts.append(
                (jax.nn.gelu(h[:, :H], approximate=True) * h[:, H:] *
                 scale).astype(jnp.bfloat16))
        for k in range(TPS):
            ys_ref[k * BT:(k + 1) * BT, :] = lax.dot_general(
                acts[k], lin_refs[k][0].astype(jnp.bfloat16),
                (((1,), (0,)), ((), ())),
                preferred_element_type=jnp.float32)


def _route(x2, router_scale, router_logits):
    dst2, te2, xb = pl.pallas_call(
        _router_body,
        out_shape=[
            jax.ShapeDtypeStruct((N // 128, 128), jnp.int32),
            jax.ShapeDtypeStruct((1, TE_LEN), jnp.int32),
            jax.ShapeDtypeStruct((N, F), jnp.bfloat16),
        ],
    )(x2, router_scale.reshape(1, F), router_logits)
    return dst2.reshape(N), te2.reshape(TE_LEN), xb


def _ffn(te, xs, gw, lin, scale):
    def _wk(k):
        return lambda j, m: (m[jnp.minimum(TPS * j + k, m[0] - 1) + 1], 0, 0)

    grid_spec = pltpu.PrefetchScalarGridSpec(
        num_scalar_prefetch=1,
        grid=(MAX_TILES // TPS,),
        in_specs=[
            pl.BlockSpec((TPS * BT, F),
                         lambda j, m: (jnp.minimum(j, (m[0] - 1) // TPS), 0)),
        ] + [
            pl.BlockSpec((1, 2 * H, F), _wk(k)) for k in range(TPS)
        ] + [
            pl.BlockSpec((1, H, F), _wk(k)) for k in range(TPS)
        ] + [
            pl.BlockSpec((1, E), lambda j, m: (0, 0)),
        ],
        out_specs=pl.BlockSpec((TPS * BT, F),
                               lambda j, m: (jnp.minimum(j, (m[0] - 1) // TPS),
                                             0)),
    )
    return pl.pallas_call(
        _ffn_body,
        grid_spec=grid_spec,
        out_shape=jax.ShapeDtypeStruct((PADDED, F), jnp.float32),
    )(te, xs, *([gw] * TPS), *([lin] * TPS), scale.reshape(1, E))


_NW = 32
_ROWS_W = N // _NW


@functools.lru_cache(maxsize=None)
def _sc_kernels():
    mesh = plsc.VectorSubcoreMesh(core_axis_name="c", subcore_axis_name="s")
    scratch = [
        pltpu.VMEM((_ROWS_W,), jnp.int32),
        pltpu.VMEM((_ROWS_W, F), jnp.float32),
        pltpu.SemaphoreType.DMA,
    ]
    scratch_bf = [
        pltpu.VMEM((_ROWS_W,), jnp.int32),
        pltpu.VMEM((_ROWS_W, F // 2), jnp.float32),
        pltpu.SemaphoreType.DMA,
    ]

    @functools.partial(
        pl.kernel, mesh=mesh,
        out_type=jax.ShapeDtypeStruct((PADDED, F // 2), jnp.float32),
        scratch_types=scratch_bf,
    )
    def sc_scatter(x_hbm, dst_hbm, xs_hbm, idx_v, rows_v, sem):
        wid = lax.axis_index("s") * 2 + lax.axis_index("c")
        base = wid * _ROWS_W
        pltpu.sync_copy(x_hbm.at[pl.ds(base, _ROWS_W)], rows_v)
        pltpu.sync_copy(dst_hbm.at[pl.ds(base, _ROWS_W)], idx_v)
        pltpu.async_copy(rows_v, xs_hbm.at[idx_v], sem).wait()

    @functools.partial(
        pl.kernel, mesh=mesh,
        out_type=jax.ShapeDtypeStruct((N, F), jnp.float32),
        scratch_types=scratch,
    )
    def sc_gather(ys_hbm, dst_hbm, out_hbm, idx_v, rows_v, sem):
        wid = lax.axis_index("s") * 2 + lax.axis_index("c")
        base = wid * _ROWS_W
        pltpu.sync_copy(dst_hbm.at[pl.ds(base, _ROWS_W)], idx_v)
        pltpu.async_copy(ys_hbm.at[idx_v], rows_v, sem).wait()
        pltpu.sync_copy(rows_v, out_hbm.at[pl.ds(base, _ROWS_W)])

    return sc_scatter, sc_gather


def kernel(x, router_scale, router_logits, gating_einsum, linear,
           per_expert_scale):
    B, L, D = x.shape
    x2 = x.reshape(B * L, D)
    dst, te, xb = _route(x2, router_scale, router_logits)
    sc_scatter, sc_gather = _sc_kernels()
    xb32 = lax.bitcast_convert_type(xb.reshape(N, F // 2, 2), jnp.float32)
    xs32 = sc_scatter(xb32, dst)
    xs = lax.bitcast_convert_type(xs32, jnp.bfloat16).reshape(PADDED, F)
    gw = gating_einsum.reshape(E, 2 * H, F)
    ys = _ffn(te, xs, gw, linear, per_expert_scale)
    out = sc_gather(ys, dst)
    return out.reshape(B, L, D)


# speedup vs baseline: 3.2759x; 3.2759x over previous
import functools
import jax
import jax.numpy as jnp
from jax import lax
from jax.experimental import pallas as pl
from jax.experimental.pallas import tpu as pltpu
from jax.experimental.pallas import tpu_sc as plsc

F = 768
H = 64
E = 64
N = 2048
BT = 64
MAX_TILES = 96
TE_LEN = 128
PADDED = MAX_TILES * BT


def _router_body(x_ref, rs_ref, rl_ref, dst_ref, te_ref):
    xv = x_ref[...]
    var = jnp.mean(xv * xv, axis=1, keepdims=True)
    ri = xv * lax.rsqrt(var + 1e-6)
    ri = ri * lax.rsqrt(jnp.float32(F)) * rs_ref[...]
    logits = jnp.dot(ri, rl_ref[...], preferred_element_type=jnp.float32)
    lane = lax.broadcasted_iota(jnp.int32, logits.shape, 1)
    maxv = jnp.max(logits, axis=1, keepdims=True)
    eid = jnp.min(jnp.where(logits == maxv, lane, E), axis=1)
    onehot = (eid[:, None] == lane).astype(jnp.float32)
    CH = 128
    NC = N // CH
    row = lax.broadcasted_iota(jnp.int32, (CH, CH), 0)
    col = lax.broadcasted_iota(jnp.int32, (CH, CH), 1)
    ltri = (row >= col).astype(jnp.float32)
    chunk_cums = []
    chunk_tot = []
    for ci in range(NC):
        blk = onehot[ci * CH:(ci + 1) * CH, :]
        cc = lax.dot_general(ltri, blk, (((1,), (0,)), ((), ())),
                             preferred_element_type=jnp.float32)
        chunk_cums.append(cc)
        chunk_tot.append(cc[CH - 1:CH, :])
    tot = jnp.concatenate(chunk_tot, axis=0)
    inc = tot
    k = 1
    while k < NC:
        inc = inc + jnp.concatenate(
            [jnp.zeros((k, E), jnp.float32), inc[: NC - k]], axis=0)
        k *= 2
    excl = inc - tot
    c = jnp.concatenate(
        [chunk_cums[ci] + excl[ci:ci + 1, :] for ci in range(NC)], axis=0)
    c = c.astype(jnp.int32)
    onehot = onehot.astype(jnp.int32)
    counts = c[N - 1 : N, :]
    rank = jnp.sum(onehot * c, axis=1) - 1
    pc = ((counts + BT - 1) // BT) * BT
    pci = pc
    k = 1
    while k < E:
        pci = pci + jnp.concatenate(
            [jnp.zeros((1, k), jnp.int32), pci[:, : E - k]], axis=1)
        k *= 2
    po = pci - pc
    dst = jnp.sum(onehot * po, axis=1) + rank
    dst_ref[...] = dst.reshape(N // 128, 128)
    starts = lax.broadcasted_iota(jnp.int32, (TE_LEN, E), 0) * BT
    te = jnp.sum((pci <= starts).astype(jnp.int32), axis=1)
    te = jnp.minimum(te, E - 1).reshape(1, TE_LEN)
    total = jnp.sum(jnp.where(lane[:1, :] == E - 1, pci, 0))
    n_used = total // BT
    meta_idx = lax.broadcasted_iota(jnp.int32, (1, TE_LEN), 1)
    te_shift = jnp.concatenate([te[:, -1:], te[:, :-1]], axis=1)
    te_ref[...] = jnp.where(meta_idx == 0, n_used, te_shift)


def _ffn_tile(xt, gw, lin, e, sc_ref):
    xb = xt.astype(jnp.bfloat16)
    h = lax.dot_general(xb, gw.astype(jnp.bfloat16),
                        (((1,), (1,)), ((), ())),
                        preferred_element_type=jnp.float32)
    lane = lax.broadcasted_iota(jnp.int32, (1, E), 1)
    scale = jnp.sum(jnp.where(lane == e, sc_ref[...], 0.0))
    act = jax.nn.gelu(h[:, :H], approximate=True) * h[:, H:] * scale
    return lax.dot_general(act.astype(jnp.bfloat16), lin.astype(jnp.bfloat16),
                           (((1,), (0,)), ((), ())),
                           preferred_element_type=jnp.float32)


TPS = 8


def _ffn_body(meta_ref, xs_ref, *refs):
    gw_refs = refs[:TPS]
    lin_refs = refs[TPS:2 * TPS]
    sc_ref = refs[2 * TPS]
    ys_ref = refs[2 * TPS + 1]
    ja = TPS * pl.program_id(0)
    n_used = meta_ref[0]

    @pl.when(ja < n_used)
    def _():
        lane = lax.broadcasted_iota(jnp.int32, (1, E), 1)
        hs = []
        for k in range(TPS):
            xb = xs_ref[k * BT:(k + 1) * BT, :].astype(jnp.bfloat16)
            hs.append(lax.dot_general(
                xb, gw_refs[k][0].astype(jnp.bfloat16),
                (((1,), (1,)), ((), ())),
                preferred_element_type=jnp.float32))
        acts = []
        for k in range(TPS):
            e = meta_ref[jnp.minimum(ja + k, n_used - 1) + 1]
            scale = jnp.sum(jnp.where(lane == e, sc_ref[...], 0.0))
            h = hs[k]
            acts.append(
                (jax.nn.gelu(h[:, :H], approximate=True) * h[:, H:] *
                 scale).astype(jnp.bfloat16))
        for k in range(TPS):
            ys_ref[k * BT:(k + 1) * BT, :] = lax.dot_general(
                acts[k], lin_refs[k][0].astype(jnp.bfloat16),
                (((1,), (0,)), ((), ())),
                preferred_element_type=jnp.float32)


def _route(x2, router_scale, router_logits):
    dst2, te2 = pl.pallas_call(
        _router_body,
        out_shape=[
            jax.ShapeDtypeStruct((N // 128, 128), jnp.int32),
            jax.ShapeDtypeStruct((1, TE_LEN), jnp.int32),
        ],
    )(x2, router_scale.reshape(1, F), router_logits)
    return dst2.reshape(N), te2.reshape(TE_LEN)


def _ffn(te, xs, gw, lin, scale):
    def _wk(k):
        return lambda j, m: (m[jnp.minimum(TPS * j + k, m[0] - 1) + 1], 0, 0)

    grid_spec = pltpu.PrefetchScalarGridSpec(
        num_scalar_prefetch=1,
        grid=(MAX_TILES // TPS,),
        in_specs=[
            pl.BlockSpec((TPS * BT, F),
                         lambda j, m: (jnp.minimum(j, (m[0] - 1) // TPS), 0)),
        ] + [
            pl.BlockSpec((1, 2 * H, F), _wk(k)) for k in range(TPS)
        ] + [
            pl.BlockSpec((1, H, F), _wk(k)) for k in range(TPS)
        ] + [
            pl.BlockSpec((1, E), lambda j, m: (0, 0)),
        ],
        out_specs=pl.BlockSpec((TPS * BT, F),
                               lambda j, m: (jnp.minimum(j, (m[0] - 1) // TPS),
                                             0)),
    )
    return pl.pallas_call(
        _ffn_body,
        grid_spec=grid_spec,
        out_shape=jax.ShapeDtypeStruct((PADDED, F), jnp.float32),
    )(te, xs, *([gw] * TPS), *([lin] * TPS), scale.reshape(1, E))


_NW = 32
_ROWS_W = N // _NW


@functools.lru_cache(maxsize=None)
def _sc_kernels():
    mesh = plsc.VectorSubcoreMesh(core_axis_name="c", subcore_axis_name="s")
    scratch = [
        pltpu.VMEM((_ROWS_W,), jnp.int32),
        pltpu.VMEM((_ROWS_W, F), jnp.float32),
        pltpu.SemaphoreType.DMA,
    ]

    @functools.partial(
        pl.kernel, mesh=mesh,
        out_type=jax.ShapeDtypeStruct((PADDED, F), jnp.float32),
        scratch_types=scratch,
    )
    def sc_scatter(x_hbm, dst_hbm, xs_hbm, idx_v, rows_v, sem):
        wid = lax.axis_index("s") * 2 + lax.axis_index("c")
        base = wid * _ROWS_W
        pltpu.sync_copy(x_hbm.at[pl.ds(base, _ROWS_W)], rows_v)
        pltpu.sync_copy(dst_hbm.at[pl.ds(base, _ROWS_W)], idx_v)
        pltpu.async_copy(rows_v, xs_hbm.at[idx_v], sem).wait()

    @functools.partial(
        pl.kernel, mesh=mesh,
        out_type=jax.ShapeDtypeStruct((N, F), jnp.float32),
        scratch_types=scratch,
    )
    def sc_gather(ys_hbm, dst_hbm, out_hbm, idx_v, rows_v, sem):
        wid = lax.axis_index("s") * 2 + lax.axis_index("c")
        base = wid * _ROWS_W
        pltpu.sync_copy(dst_hbm.at[pl.ds(base, _ROWS_W)], idx_v)
        pltpu.async_copy(ys_hbm.at[idx_v], rows_v, sem).wait()
        pltpu.sync_copy(rows_v, out_hbm.at[pl.ds(base, _ROWS_W)])

    return sc_scatter, sc_gather


def kernel(x, router_scale, router_logits, gating_einsum, linear,
           per_expert_scale):
    B, L, D = x.shape
    x2 = x.reshape(B * L, D)
    dst, te = _route(x2, router_scale, router_logits)
    sc_scatter, sc_gather = _sc_kernels()
    xs = sc_scatter(x2, dst)
    gw = gating_einsum.reshape(E, 2 * H, F)
    ys = _ffn(te, xs, gw, linear, per_expert_scale)
    out = sc_gather(ys, dst)
    return out.reshape(B, L, D)
